# baseline (speedup 1.0000x reference)
import functools

import jax
import jax.numpy as jnp
import numpy as np
from jax import lax
from jax.experimental import pallas as pl
from jax.experimental.pallas import tpu as pltpu
from jax.experimental.pallas import tpu_sc as plsc

D = 59
IMG_C = 64
FH, FW = 32, 88
IH, IW = 256, 704
NXX, NXY, NXZ = 128, 128, 1
NPIX = FH * FW
NVOX = NXX * NXY
CHUNK = 128
NCHUNK = NPIX // CHUNK
DX_NP = np.array([0.8, 0.8, 20.0], dtype=np.float32)
BX_NP = np.array([-50.8, -50.8, 0.0], dtype=np.float32)

_NC, _NS = 2, 16
_ROWS_PER_TILE = NVOX // _NS


def _conv2d(x, w, b=None, stride=1, pad=0):
    out = lax.conv_general_dilated(
        x, w, (stride, stride), ((pad, pad), (pad, pad)),
        dimension_numbers=('NCHW', 'OIHW', 'NCHW'))
    if b is not None:
        out = out + b[None, :, None, None]
    return out


def _create_frustum():
    ds = jnp.broadcast_to(jnp.arange(1.0, 60.0, 1.0, dtype=jnp.float32).reshape(-1, 1, 1), (D, FH, FW))
    xs = jnp.broadcast_to(jnp.linspace(0.0, IW - 1.0, FW, dtype=jnp.float32).reshape(1, 1, FW), (D, FH, FW))
    ys = jnp.broadcast_to(jnp.linspace(0.0, IH - 1.0, FH, dtype=jnp.float32).reshape(1, FH, 1), (D, FH, FW))
    return jnp.stack([xs, ys, ds], -1)


_NPAIR = NVOX // 2
_PROWS_PER_TILE = _NPAIR // _NS
_GROWS = CHUNK // 2
_SLABP = 24


def _sc_bev_pool_body(dep_hbm, vox_hbm, feat_hbm, zeros_hbm, out_hbm,
                      w_s, vox_s, gfeat, sfeat, idxbuf, bev_sh):
    cid = lax.axis_index("c")
    sid = lax.axis_index("s")
    pltpu.sync_copy(zeros_hbm.at[pl.ds(sid * _PROWS_PER_TILE, _PROWS_PER_TILE)],
                    bev_sh.at[pl.ds(sid * _PROWS_PER_TILE, _PROWS_PER_TILE)])
    plsc.subcore_barrier()

    wid = cid * _NS + sid
    zero16 = jnp.zeros((16,), jnp.float32)

    def chunk_body(view, slab, c):
        pltpu.sync_copy(feat_hbm.at[pl.ds(view * (NPIX // 2) + c * _GROWS, _GROWS)],
                        gfeat)

        def blk_body(q, _):
            v16 = vox_s[c, pl.ds(q * 16, 16)]
            w16 = w_s[c, pl.ds(q * 16, 16)]
            idxbuf[pl.ds(q * 16, 16)] = lax.shift_right_logical(v16, 1)
            for j in range(16):
                p_row = q * 8 + j // 2
                half = (j % 2) * 64
                wj = w16[j]
                even = (v16[j] & 1) == 0
                for g in range(IMG_C // 16):
                    sc_g = gfeat[p_row, pl.ds(half + g * 16, 16)] * wj
                    sfeat[q * 16 + j, pl.ds(g * 16, 16)] = jnp.where(even, sc_g, zero16)
                    sfeat[q * 16 + j, pl.ds(64 + g * 16, 16)] = jnp.where(even, zero16, sc_g)
            return 0

        lax.fori_loop(0, CHUNK // 16, blk_body, 0)
        pltpu.sync_copy(sfeat, bev_sh.at[idxbuf], add=True)

    for view in range(6):
        def slab_body(i, _, view=view):
            d = wid + i * (_NC * _NS)

            @pl.when(d < D)
            def _():
                slab = view * D + d
                pltpu.sync_copy(dep_hbm.at[pl.ds(slab * _SLABP, _SLABP)], w_s)
                pltpu.sync_copy(vox_hbm.at[pl.ds(slab * _SLABP, _SLABP)], vox_s)

                def cbody(c, _):
                    chunk_body(view, slab, c)
                    return 0
                lax.fori_loop(0, NCHUNK, cbody, 0)
            return 0
        lax.fori_loop(0, 2, slab_body, 0)

    plsc.subcore_barrier()
    orow = cid * _NPAIR + sid * _PROWS_PER_TILE
    pltpu.sync_copy(bev_sh.at[pl.ds(sid * _PROWS_PER_TILE, _PROWS_PER_TILE)],
                    out_hbm.at[pl.ds(orow, _PROWS_PER_TILE)])


@jax.jit
def _sc_bev_pool(dep, vox, feat, zeros):
    mesh = plsc.VectorSubcoreMesh(core_axis_name="c", subcore_axis_name="s")
    out = pl.kernel(
        _sc_bev_pool_body,
        out_type=jax.ShapeDtypeStruct((_NC * _NPAIR, 128), jnp.float32),
        mesh=mesh,
        scratch_types=[
            pltpu.VMEM((_SLABP, CHUNK), jnp.float32),
            pltpu.VMEM((_SLABP, CHUNK), jnp.int32),
            pltpu.VMEM((_GROWS, 128), jnp.float32),
            pltpu.VMEM((CHUNK, 128), jnp.float32),
            pltpu.VMEM((CHUNK,), jnp.int32),
            pltpu.VMEM_SHARED((_NPAIR, 128), jnp.float32),
        ],
    )(jnp.pad(dep.reshape(6 * D, NCHUNK, CHUNK), ((0, 0), (0, _SLABP - NCHUNK), (0, 0))).reshape(6 * D * _SLABP, CHUNK),
      jnp.pad(vox.reshape(6 * D, NCHUNK, CHUNK), ((0, 0), (0, _SLABP - NCHUNK), (0, 0))).reshape(6 * D * _SLABP, CHUNK),
      feat.reshape(6 * NPIX // 2, 128),
      zeros)
    return out.reshape(_NC, NVOX, IMG_C)


def _dn_body(x_ref, w1_ref, b1_ref, w2_ref, b2_ref, w3_ref, b3_ref,
             dep_ref, feat_ref, h1pad):
    f32 = jnp.float32
    bf16 = jnp.bfloat16
    x = x_ref[0]
    acc = jnp.zeros((FH * FW, 256), f32)
    for dy in range(3):
        for dx in range(3):
            a = x[dy:dy + FH, dx:dx + FW, :].reshape(FH * FW, 320)
            acc += jnp.dot(a.astype(bf16), w1_ref[dy, dx].astype(bf16),
                           preferred_element_type=f32)
    h1 = jax.nn.relu(acc + b1_ref[0])
    h1pad[...] = jnp.zeros((FH + 2, FW + 2, 256), f32)
    h1pad[1:FH + 1, 1:FW + 1, :] = h1.reshape(FH, FW, 256)
    hp = h1pad[...]
    acc2 = jnp.zeros((FH * FW, 256), f32)
    for dy in range(3):
        for dx in range(3):
            a = hp[dy:dy + FH, dx:dx + FW, :].reshape(FH * FW, 256)
            acc2 += jnp.dot(a.astype(bf16), w2_ref[dy, dx].astype(bf16),
                            preferred_element_type=f32)
    h2 = jax.nn.relu(acc2 + b2_ref[0])
    x3 = jnp.dot(h2.astype(bf16), w3_ref[...].astype(bf16),
                 preferred_element_type=f32) + b3_ref[0]
    logits = x3[:, :D]
    m = jnp.max(logits, axis=1, keepdims=True)
    e = jnp.exp(logits - m)
    dep = e / jnp.sum(e, axis=1, keepdims=True)
    dep_ref[0] = dep.T
    feat_ref[0] = x3[:, D:D + IMG_C]


@jax.jit
def _dn_chain(xpad, w1, b1, w2, b2, w3, b3):
    return pl.pallas_call(
        _dn_body,
        grid=(6,),
        in_specs=[
            pl.BlockSpec((1, FH + 2, FW + 2, 320), lambda v: (v, 0, 0, 0)),
            pl.BlockSpec((3, 3, 320, 256), lambda v: (0, 0, 0, 0)),
            pl.BlockSpec((1, 256), lambda v: (0, 0)),
            pl.BlockSpec((3, 3, 256, 256), lambda v: (0, 0, 0, 0)),
            pl.BlockSpec((1, 256), lambda v: (0, 0)),
            pl.BlockSpec((256, 128), lambda v: (0, 0)),
            pl.BlockSpec((1, 128), lambda v: (0, 0)),
        ],
        out_specs=[
            pl.BlockSpec((1, D, NPIX), lambda v: (v, 0, 0)),
            pl.BlockSpec((1, NPIX, IMG_C), lambda v: (v, 0, 0)),
        ],
        out_shape=[
            jax.ShapeDtypeStruct((6, D, NPIX), jnp.float32),
            jax.ShapeDtypeStruct((6, NPIX, IMG_C), jnp.float32),
        ],
        scratch_shapes=[pltpu.VMEM((FH + 2, FW + 2, 256), jnp.float32)],
    )(xpad, w1, b1, w2, b2, w3, b3)


_FT = 16


def _final_body(xp_ref, wf_ref, out_ref):
    f32 = jnp.float32
    bf16 = jnp.bfloat16
    t = pl.program_id(0)
    xt = xp_ref[0, pl.ds(t * _FT, _FT + 2), :, :] + \
        xp_ref[1, pl.ds(t * _FT, _FT + 2), :, :]
    acc = jnp.zeros((_FT * NXY, IMG_C), f32)
    for dy in range(3):
        for dx in range(3):
            a = xt[dy:dy + _FT, dx:dx + NXY, :].reshape(_FT * NXY, IMG_C)
            acc += jnp.dot(a.astype(bf16), wf_ref[dy, dx].astype(bf16),
                           preferred_element_type=f32)
    out_ref[...] = jax.nn.relu(acc)


@jax.jit
def _final_conv(parts, wf):
    xp = jnp.pad(parts.reshape(2, NXX, NXY, IMG_C),
                 ((0, 0), (1, 1), (1, 1), (0, 0)))
    return pl.pallas_call(
        _final_body,
        grid=(NXX // _FT,),
        in_specs=[
            pl.BlockSpec((2, NXX + 2, NXY + 2, IMG_C), lambda t: (0, 0, 0, 0)),
            pl.BlockSpec((3, 3, IMG_C, IMG_C), lambda t: (0, 0, 0, 0)),
        ],
        out_specs=pl.BlockSpec((_FT * NXY, IMG_C), lambda t: (t, 0)),
        out_shape=jax.ShapeDtypeStruct((NVOX, IMG_C), jnp.float32),
    )(xp, wf)


def kernel(image_fpn, depth, camera_intrinsics, camera2lidar, img_aug_matrix,
           lidar_aug_matrix, dt_w1, dt_b1, dt_w2, dt_b2, dt_w3, dt_b3,
           dn_w1, dn_b1, dn_w2, dn_b2, dn_w3, dn_b3, ds_w1):
    relu = jax.nn.relu
    d = depth.reshape(6, 1, IH, IW)
    d = relu(_conv2d(d, dt_w1, dt_b1, 1, 0))
    d = relu(_conv2d(d, dt_w2, dt_b2, 4, 2))
    d = relu(_conv2d(d, dt_w3, dt_b3, 2, 2))
    x = jnp.concatenate([d.transpose(0, 2, 3, 1),
                         image_fpn.transpose(0, 2, 3, 1)], axis=-1)
    xpad = jnp.pad(x, ((0, 0), (1, 1), (1, 1), (0, 0)))
    w1 = dn_w1.transpose(2, 3, 1, 0)
    w2 = dn_w2.transpose(2, 3, 1, 0)
    w3 = jnp.pad(dn_w3[:, :, 0, 0].T, ((0, 0), (0, 5)))
    b3 = jnp.pad(dn_b3, (0, 5))
    dep6, feat2d = _dn_chain(xpad, w1, dn_b1[None], w2, dn_b2[None],
                             w3, b3[None])
    dep = dep6

    intrins = camera_intrinsics[..., :3, :3]
    post_rots = img_aug_matrix[..., :3, :3]
    post_trans = img_aug_matrix[..., :3, 3]
    c2l_r = camera2lidar[..., :3, :3]
    c2l_t = camera2lidar[..., :3, 3]
    fr = _create_frustum()
    pts = fr[None, None] - post_trans[:, :, None, None, None, :]
    pts = jnp.einsum('bnij,bndhwj->bndhwi', jnp.linalg.inv(post_rots), pts)
    pts = jnp.concatenate([pts[..., :2] * pts[..., 2:3], pts[..., 2:3]], -1)
    combine = c2l_r @ jnp.linalg.inv(intrins)
    pts = jnp.einsum('bnij,bndhwj->bndhwi', combine, pts) + c2l_t[:, :, None, None, None, :]
    pts = (jnp.einsum('bij,bndhwj->bndhwi', lidar_aug_matrix[..., :3, :3], pts)
           + lidar_aug_matrix[..., :3, 3][:, None, None, None, None, :])
    g = ((pts - jnp.asarray(BX_NP - DX_NP / 2.0)) / jnp.asarray(DX_NP)).astype(jnp.int32).reshape(6, D, NPIX, 3)
    kept = ((g[..., 0] >= 0) & (g[..., 0] < NXX) & (g[..., 1] >= 0) &
            (g[..., 1] < NXY) & (g[..., 2] >= 0) & (g[..., 2] < NXZ))
    gx = jnp.where(kept, g[..., 0], 0)
    gy = jnp.where(kept, g[..., 1], 0)
    vox = gx * NXY + gy

    dep_w = dep * kept.astype(jnp.float32)
    zeros = jnp.zeros((NVOX // 2, 128), jnp.float32)

    parts = _sc_bev_pool(dep_w, vox, feat2d, zeros)

    wf = ds_w1.transpose(2, 3, 1, 0)
    res = _final_conv(parts, wf)
    out = res.reshape(NXX, NXY, IMG_C).transpose(2, 0, 1)[None]
    return out


# speedup vs baseline: 1.0326x; 1.0326x over previous
import functools

import jax
import jax.numpy as jnp
import numpy as np
from jax import lax
from jax.experimental import pallas as pl
from jax.experimental.pallas import tpu as pltpu
from jax.experimental.pallas import tpu_sc as plsc

D = 59
IMG_C = 64
FH, FW = 32, 88
IH, IW = 256, 704
NXX, NXY, NXZ = 128, 128, 1
NPIX = FH * FW
NVOX = NXX * NXY
CHUNK = 128
NCHUNK = NPIX // CHUNK
DX_NP = np.array([0.8, 0.8, 20.0], dtype=np.float32)
BX_NP = np.array([-50.8, -50.8, 0.0], dtype=np.float32)

_NC, _NS = 2, 16
_ROWS_PER_TILE = NVOX // _NS


def _conv2d(x, w, b=None, stride=1, pad=0):
    out = lax.conv_general_dilated(
        x, w, (stride, stride), ((pad, pad), (pad, pad)),
        dimension_numbers=('NCHW', 'OIHW', 'NCHW'))
    if b is not None:
        out = out + b[None, :, None, None]
    return out


def _create_frustum():
    ds = jnp.broadcast_to(jnp.arange(1.0, 60.0, 1.0, dtype=jnp.float32).reshape(-1, 1, 1), (D, FH, FW))
    xs = jnp.broadcast_to(jnp.linspace(0.0, IW - 1.0, FW, dtype=jnp.float32).reshape(1, 1, FW), (D, FH, FW))
    ys = jnp.broadcast_to(jnp.linspace(0.0, IH - 1.0, FH, dtype=jnp.float32).reshape(1, FH, 1), (D, FH, FW))
    return jnp.stack([xs, ys, ds], -1)


_NPAIR = NVOX // 2
_PROWS_PER_TILE = _NPAIR // _NS
_GROWS = CHUNK // 2
_SLABP = 24


def _sc_bev_pool_body(dep_hbm, vox_hbm, feat_hbm, zeros_hbm, out_hbm,
                      w_s, vox_s, gfeat, sfeat, idxbuf, bev_sh):
    cid = lax.axis_index("c")
    sid = lax.axis_index("s")
    pltpu.sync_copy(zeros_hbm.at[pl.ds(sid * _PROWS_PER_TILE, _PROWS_PER_TILE)],
                    bev_sh.at[pl.ds(sid * _PROWS_PER_TILE, _PROWS_PER_TILE)])
    plsc.subcore_barrier()

    wid = cid * _NS + sid
    zero16 = jnp.zeros((16,), jnp.float32)

    def chunk_body(view, slab, c):
        pltpu.sync_copy(feat_hbm.at[pl.ds(view * (NPIX // 2) + c * _GROWS, _GROWS)],
                        gfeat)

        def blk_body(q, _):
            v16 = vox_s[c, pl.ds(q * 16, 16)]
            w16 = w_s[c, pl.ds(q * 16, 16)]
            idxbuf[pl.ds(q * 16, 16)] = lax.shift_right_logical(v16, 1)
            for j in range(16):
                p_row = q * 8 + j // 2
                half = (j % 2) * 64
                wj = w16[j]
                base = (v16[j] & 1) * 64
                for g in range(IMG_C // 16):
                    sc_g = gfeat[p_row, pl.ds(half + g * 16, 16)] * wj
                    sfeat[q * 16 + j, pl.ds(base + g * 16, 16)] = sc_g
                    sfeat[q * 16 + j, pl.ds((64 - base) + g * 16, 16)] = zero16
            return 0

        lax.fori_loop(0, CHUNK // 16, blk_body, 0)
        pltpu.sync_copy(sfeat, bev_sh.at[idxbuf], add=True)

    for view in range(6):
        def slab_body(i, _, view=view):
            d = wid + i * (_NC * _NS)

            @pl.when(d < D)
            def _():
                slab = view * D + d
                pltpu.sync_copy(dep_hbm.at[pl.ds(slab * _SLABP, _SLABP)], w_s)
                pltpu.sync_copy(vox_hbm.at[pl.ds(slab * _SLABP, _SLABP)], vox_s)

                def cbody(c, _):
                    chunk_body(view, slab, c)
                    return 0
                lax.fori_loop(0, NCHUNK, cbody, 0)
            return 0
        lax.fori_loop(0, 2, slab_body, 0)

    plsc.subcore_barrier()
    orow = cid * _NPAIR + sid * _PROWS_PER_TILE
    pltpu.sync_copy(bev_sh.at[pl.ds(sid * _PROWS_PER_TILE, _PROWS_PER_TILE)],
                    out_hbm.at[pl.ds(orow, _PROWS_PER_TILE)])


@jax.jit
def _sc_bev_pool(dep, vox, feat, zeros):
    mesh = plsc.VectorSubcoreMesh(core_axis_name="c", subcore_axis_name="s")
    out = pl.kernel(
        _sc_bev_pool_body,
        out_type=jax.ShapeDtypeStruct((_NC * _NPAIR, 128), jnp.float32),
        mesh=mesh,
        scratch_types=[
            pltpu.VMEM((_SLABP, CHUNK), jnp.float32),
            pltpu.VMEM((_SLABP, CHUNK), jnp.int32),
            pltpu.VMEM((_GROWS, 128), jnp.float32),
            pltpu.VMEM((CHUNK, 128), jnp.float32),
            pltpu.VMEM((CHUNK,), jnp.int32),
            pltpu.VMEM_SHARED((_NPAIR, 128), jnp.float32),
        ],
    )(jnp.pad(dep.reshape(6 * D, NCHUNK, CHUNK), ((0, 0), (0, _SLABP - NCHUNK), (0, 0))).reshape(6 * D * _SLABP, CHUNK),
      jnp.pad(vox.reshape(6 * D, NCHUNK, CHUNK), ((0, 0), (0, _SLABP - NCHUNK), (0, 0))).reshape(6 * D * _SLABP, CHUNK),
      feat.reshape(6 * NPIX // 2, 128),
      zeros)
    return out.reshape(_NC, NVOX, IMG_C)


def _dn_body(x_ref, w1_ref, b1_ref, w2_ref, b2_ref, w3_ref, b3_ref,
             dep_ref, feat_ref, h1pad):
    f32 = jnp.float32
    bf16 = jnp.bfloat16
    x = x_ref[0]
    acc = jnp.zeros((FH * FW, 256), f32)
    for dy in range(3):
        for dx in range(3):
            a = x[dy:dy + FH, dx:dx + FW, :].reshape(FH * FW, 320)
            acc += jnp.dot(a.astype(bf16), w1_ref[dy, dx].astype(bf16),
                           preferred_element_type=f32)
    h1 = jax.nn.relu(acc + b1_ref[0])
    h1pad[...] = jnp.zeros((FH + 2, FW + 2, 256), f32)
    h1pad[1:FH + 1, 1:FW + 1, :] = h1.reshape(FH, FW, 256)
    hp = h1pad[...]
    acc2 = jnp.zeros((FH * FW, 256), f32)
    for dy in range(3):
        for dx in range(3):
            a = hp[dy:dy + FH, dx:dx + FW, :].reshape(FH * FW, 256)
            acc2 += jnp.dot(a.astype(bf16), w2_ref[dy, dx].astype(bf16),
                            preferred_element_type=f32)
    h2 = jax.nn.relu(acc2 + b2_ref[0])
    x3 = jnp.dot(h2.astype(bf16), w3_ref[...].astype(bf16),
                 preferred_element_type=f32) + b3_ref[0]
    logits = x3[:, :D]
    m = jnp.max(logits, axis=1, keepdims=True)
    e = jnp.exp(logits - m)
    dep = e / jnp.sum(e, axis=1, keepdims=True)
    dep_ref[0] = dep.T
    feat_ref[0] = x3[:, D:D + IMG_C]


@jax.jit
def _dn_chain(xpad, w1, b1, w2, b2, w3, b3):
    return pl.pallas_call(
        _dn_body,
        grid=(6,),
        in_specs=[
            pl.BlockSpec((1, FH + 2, FW + 2, 320), lambda v: (v, 0, 0, 0)),
            pl.BlockSpec((3, 3, 320, 256), lambda v: (0, 0, 0, 0)),
            pl.BlockSpec((1, 256), lambda v: (0, 0)),
            pl.BlockSpec((3, 3, 256, 256), lambda v: (0, 0, 0, 0)),
            pl.BlockSpec((1, 256), lambda v: (0, 0)),
            pl.BlockSpec((256, 128), lambda v: (0, 0)),
            pl.BlockSpec((1, 128), lambda v: (0, 0)),
        ],
        out_specs=[
            pl.BlockSpec((1, D, NPIX), lambda v: (v, 0, 0)),
            pl.BlockSpec((1, NPIX, IMG_C), lambda v: (v, 0, 0)),
        ],
        out_shape=[
            jax.ShapeDtypeStruct((6, D, NPIX), jnp.float32),
            jax.ShapeDtypeStruct((6, NPIX, IMG_C), jnp.float32),
        ],
        scratch_shapes=[pltpu.VMEM((FH + 2, FW + 2, 256), jnp.float32)],
    )(xpad, w1, b1, w2, b2, w3, b3)


_FT = 16


def _final_body(xp_ref, wf_ref, out_ref):
    f32 = jnp.float32
    bf16 = jnp.bfloat16
    t = pl.program_id(0)
    xt = xp_ref[0, pl.ds(t * _FT, _FT + 2), :, :] + \
        xp_ref[1, pl.ds(t * _FT, _FT + 2), :, :]
    acc = jnp.zeros((_FT * NXY, IMG_C), f32)
    for dy in range(3):
        for dx in range(3):
            a = xt[dy:dy + _FT, dx:dx + NXY, :].reshape(_FT * NXY, IMG_C)
            acc += jnp.dot(a.astype(bf16), wf_ref[dy, dx].astype(bf16),
                           preferred_element_type=f32)
    out_ref[...] = jax.nn.relu(acc)


@jax.jit
def _final_conv(parts, wf):
    xp = jnp.pad(parts.reshape(2, NXX, NXY, IMG_C),
                 ((0, 0), (1, 1), (1, 1), (0, 0)))
    return pl.pallas_call(
        _final_body,
        grid=(NXX // _FT,),
        in_specs=[
            pl.BlockSpec((2, NXX + 2, NXY + 2, IMG_C), lambda t: (0, 0, 0, 0)),
            pl.BlockSpec((3, 3, IMG_C, IMG_C), lambda t: (0, 0, 0, 0)),
        ],
        out_specs=pl.BlockSpec((_FT * NXY, IMG_C), lambda t: (t, 0)),
        out_shape=jax.ShapeDtypeStruct((NVOX, IMG_C), jnp.float32),
    )(xp, wf)


def kernel(image_fpn, depth, camera_intrinsics, camera2lidar, img_aug_matrix,
           lidar_aug_matrix, dt_w1, dt_b1, dt_w2, dt_b2, dt_w3, dt_b3,
           dn_w1, dn_b1, dn_w2, dn_b2, dn_w3, dn_b3, ds_w1):
    relu = jax.nn.relu
    d = depth.reshape(6, 1, IH, IW)
    d = relu(_conv2d(d, dt_w1, dt_b1, 1, 0))
    d = relu(_conv2d(d, dt_w2, dt_b2, 4, 2))
    d = relu(_conv2d(d, dt_w3, dt_b3, 2, 2))
    x = jnp.concatenate([d.transpose(0, 2, 3, 1),
                         image_fpn.transpose(0, 2, 3, 1)], axis=-1)
    xpad = jnp.pad(x, ((0, 0), (1, 1), (1, 1), (0, 0)))
    w1 = dn_w1.transpose(2, 3, 1, 0)
    w2 = dn_w2.transpose(2, 3, 1, 0)
    w3 = jnp.pad(dn_w3[:, :, 0, 0].T, ((0, 0), (0, 5)))
    b3 = jnp.pad(dn_b3, (0, 5))
    dep6, feat2d = _dn_chain(xpad, w1, dn_b1[None], w2, dn_b2[None],
                             w3, b3[None])
    dep = dep6

    intrins = camera_intrinsics[..., :3, :3]
    post_rots = img_aug_matrix[..., :3, :3]
    post_trans = img_aug_matrix[..., :3, 3]
    c2l_r = camera2lidar[..., :3, :3]
    c2l_t = camera2lidar[..., :3, 3]
    fr = _create_frustum()
    pts = fr[None, None] - post_trans[:, :, None, None, None, :]
    pts = jnp.einsum('bnij,bndhwj->bndhwi', jnp.linalg.inv(post_rots), pts)
    pts = jnp.concatenate([pts[..., :2] * pts[..., 2:3], pts[..., 2:3]], -1)
    combine = c2l_r @ jnp.linalg.inv(intrins)
    pts = jnp.einsum('bnij,bndhwj->bndhwi', combine, pts) + c2l_t[:, :, None, None, None, :]
    pts = (jnp.einsum('bij,bndhwj->bndhwi', lidar_aug_matrix[..., :3, :3], pts)
           + lidar_aug_matrix[..., :3, 3][:, None, None, None, None, :])
    g = ((pts - jnp.asarray(BX_NP - DX_NP / 2.0)) / jnp.asarray(DX_NP)).astype(jnp.int32).reshape(6, D, NPIX, 3)
    kept = ((g[..., 0] >= 0) & (g[..., 0] < NXX) & (g[..., 1] >= 0) &
            (g[..., 1] < NXY) & (g[..., 2] >= 0) & (g[..., 2] < NXZ))
    gx = jnp.where(kept, g[..., 0], 0)
    gy = jnp.where(kept, g[..., 1], 0)
    vox = gx * NXY + gy

    dep_w = dep * kept.astype(jnp.float32)
    zeros = jnp.zeros((NVOX // 2, 128), jnp.float32)

    parts = _sc_bev_pool(dep_w, vox, feat2d, zeros)

    wf = ds_w1.transpose(2, 3, 1, 0)
    res = _final_conv(parts, wf)
    out = res.reshape(NXX, NXY, IMG_C).transpose(2, 0, 1)[None]
    return out


# speedup vs baseline: 1.2435x; 1.2043x over previous
import functools

import jax
import jax.numpy as jnp
import numpy as np
from jax import lax
from jax.experimental import pallas as pl
from jax.experimental.pallas import tpu as pltpu
from jax.experimental.pallas import tpu_sc as plsc

D = 59
IMG_C = 64
FH, FW = 32, 88
IH, IW = 256, 704
NXX, NXY, NXZ = 128, 128, 1
NPIX = FH * FW
NVOX = NXX * NXY
CHUNK = 128
NCHUNK = NPIX // CHUNK
DX_NP = np.array([0.8, 0.8, 20.0], dtype=np.float32)
BX_NP = np.array([-50.8, -50.8, 0.0], dtype=np.float32)

_NC, _NS = 2, 16
_ROWS_PER_TILE = NVOX // _NS


def _conv2d(x, w, b=None, stride=1, pad=0):
    out = lax.conv_general_dilated(
        x, w, (stride, stride), ((pad, pad), (pad, pad)),
        dimension_numbers=('NCHW', 'OIHW', 'NCHW'))
    if b is not None:
        out = out + b[None, :, None, None]
    return out


def _create_frustum():
    ds = jnp.broadcast_to(jnp.arange(1.0, 60.0, 1.0, dtype=jnp.float32).reshape(-1, 1, 1), (D, FH, FW))
    xs = jnp.broadcast_to(jnp.linspace(0.0, IW - 1.0, FW, dtype=jnp.float32).reshape(1, 1, FW), (D, FH, FW))
    ys = jnp.broadcast_to(jnp.linspace(0.0, IH - 1.0, FH, dtype=jnp.float32).reshape(1, FH, 1), (D, FH, FW))
    return jnp.stack([xs, ys, ds], -1)


_NPAIR = NVOX // 2
_PROWS_PER_TILE = _NPAIR // _NS
_GROWS = CHUNK // 2
_SLABP = 24


def _sc_bev_pool_body(dep_hbm, vox_hbm, feat_hbm, zeros_hbm, out_hbm,
                      w_s, vox_s, gfeat, sfeat, idxbuf, bev_sh, fsem, ssem):
    cid = lax.axis_index("c")
    sid = lax.axis_index("s")
    pltpu.sync_copy(zeros_hbm.at[pl.ds(sid * _PROWS_PER_TILE, _PROWS_PER_TILE)],
                    bev_sh.at[pl.ds(sid * _PROWS_PER_TILE, _PROWS_PER_TILE)])
    plsc.subcore_barrier()

    wid = cid * _NS + sid
    zero16 = jnp.zeros((16,), jnp.float32)

    def issue_feat(view, c, buf):
        pltpu.async_copy(
            feat_hbm.at[pl.ds(view * (NPIX // 2) + c * _GROWS, _GROWS)],
            gfeat.at[pl.ds(buf * _GROWS, _GROWS)], fsem.at[buf])

    def wait_feat(buf):
        pltpu.make_async_copy(
            feat_hbm.at[pl.ds(0, _GROWS)],
            gfeat.at[pl.ds(buf * _GROWS, _GROWS)], fsem.at[buf]).wait()

    def wait_scat(buf):
        pltpu.make_async_copy(
            feat_hbm.at[pl.ds(0, CHUNK)],
            sfeat.at[pl.ds(buf * CHUNK, CHUNK)], ssem.at[buf]).wait()

    def chunk_body(c):
        buf = c & 1
        wait_feat(buf)

        def blk_body(q, _):
            v16 = vox_s[c, pl.ds(q * 16, 16)]
            w16 = w_s[c, pl.ds(q * 16, 16)]
            idxbuf[buf, pl.ds(q * 16, 16)] = lax.shift_right_logical(v16, 1)
            for j in range(16):
                p_row = buf * _GROWS + q * 8 + j // 2
                half = (j % 2) * 64
                wj = w16[j]
                base = (v16[j] & 1) * 64
                for g in range(IMG_C // 16):
                    sc_g = gfeat[p_row, pl.ds(half + g * 16, 16)] * wj
                    sfeat[buf * CHUNK + q * 16 + j, pl.ds(base + g * 16, 16)] = sc_g
                    sfeat[buf * CHUNK + q * 16 + j, pl.ds((64 - base) + g * 16, 16)] = zero16
            return 0

        lax.fori_loop(0, CHUNK // 16, blk_body, 0)
        pltpu.async_copy(sfeat.at[pl.ds(buf * CHUNK, CHUNK)],
                         bev_sh.at[idxbuf.at[buf]], ssem.at[buf], add=True)

    for view in range(6):
        def slab_body(i, _, view=view):
            d = wid + i * (_NC * _NS)

            @pl.when(d < D)
            def _():
                slab = view * D + d
                pltpu.sync_copy(dep_hbm.at[pl.ds(slab * _SLABP, _SLABP)], w_s)
                pltpu.sync_copy(vox_hbm.at[pl.ds(slab * _SLABP, _SLABP)], vox_s)
                issue_feat(view, 0, 0)

                def cbody(c, _):
                    @pl.when(c + 1 < NCHUNK)
                    def _():
                        issue_feat(view, c + 1, (c + 1) & 1)

                    @pl.when(c >= 2)
                    def _():
                        wait_scat(c & 1)
                    chunk_body(c)
                    return 0
                lax.fori_loop(0, NCHUNK, cbody, 0)
                wait_scat(0)
                wait_scat(1)
            return 0
        lax.fori_loop(0, 2, slab_body, 0)

    plsc.subcore_barrier()
    orow = cid * _NPAIR + sid * _PROWS_PER_TILE
    pltpu.sync_copy(bev_sh.at[pl.ds(sid * _PROWS_PER_TILE, _PROWS_PER_TILE)],
                    out_hbm.at[pl.ds(orow, _PROWS_PER_TILE)])


@jax.jit
def _sc_bev_pool(dep, vox, feat, zeros):
    mesh = plsc.VectorSubcoreMesh(core_axis_name="c", subcore_axis_name="s")
    out = pl.kernel(
        _sc_bev_pool_body,
        out_type=jax.ShapeDtypeStruct((_NC * _NPAIR, 128), jnp.float32),
        mesh=mesh,
        scratch_types=[
            pltpu.VMEM((_SLABP, CHUNK), jnp.float32),
            pltpu.VMEM((_SLABP, CHUNK), jnp.int32),
            pltpu.VMEM((2 * _GROWS, 128), jnp.float32),
            pltpu.VMEM((2 * CHUNK, 128), jnp.float32),
            pltpu.VMEM((2, CHUNK), jnp.int32),
            pltpu.VMEM_SHARED((_NPAIR, 128), jnp.float32),
            pltpu.SemaphoreType.DMA((2,)),
            pltpu.SemaphoreType.DMA((2,)),
        ],
    )(jnp.pad(dep.reshape(6 * D, NCHUNK, CHUNK), ((0, 0), (0, _SLABP - NCHUNK), (0, 0))).reshape(6 * D * _SLABP, CHUNK),
      jnp.pad(vox.reshape(6 * D, NCHUNK, CHUNK), ((0, 0), (0, _SLABP - NCHUNK), (0, 0))).reshape(6 * D * _SLABP, CHUNK),
      feat.reshape(6 * NPIX // 2, 128),
      zeros)
    return out.reshape(_NC, NVOX, IMG_C)


def _dn_body(x_ref, w1_ref, b1_ref, w2_ref, b2_ref, w3_ref, b3_ref,
             dep_ref, feat_ref, h1pad):
    f32 = jnp.float32
    bf16 = jnp.bfloat16
    x = x_ref[0]
    acc = jnp.zeros((FH * FW, 256), f32)
    for dy in range(3):
        for dx in range(3):
            a = x[dy:dy + FH, dx:dx + FW, :].reshape(FH * FW, 320)
            acc += jnp.dot(a.astype(bf16), w1_ref[dy, dx].astype(bf16),
                           preferred_element_type=f32)
    h1 = jax.nn.relu(acc + b1_ref[0])
    h1pad[...] = jnp.zeros((FH + 2, FW + 2, 256), f32)
    h1pad[1:FH + 1, 1:FW + 1, :] = h1.reshape(FH, FW, 256)
    hp = h1pad[...]
    acc2 = jnp.zeros((FH * FW, 256), f32)
    for dy in range(3):
        for dx in range(3):
            a = hp[dy:dy + FH, dx:dx + FW, :].reshape(FH * FW, 256)
            acc2 += jnp.dot(a.astype(bf16), w2_ref[dy, dx].astype(bf16),
                            preferred_element_type=f32)
    h2 = jax.nn.relu(acc2 + b2_ref[0])
    x3 = jnp.dot(h2.astype(bf16), w3_ref[...].astype(bf16),
                 preferred_element_type=f32) + b3_ref[0]
    logits = x3[:, :D]
    m = jnp.max(logits, axis=1, keepdims=True)
    e = jnp.exp(logits - m)
    dep = e / jnp.sum(e, axis=1, keepdims=True)
    dep_ref[0] = dep.T
    feat_ref[0] = x3[:, D:D + IMG_C]


@jax.jit
def _dn_chain(xpad, w1, b1, w2, b2, w3, b3):
    return pl.pallas_call(
        _dn_body,
        grid=(6,),
        in_specs=[
            pl.BlockSpec((1, FH + 2, FW + 2, 320), lambda v: (v, 0, 0, 0)),
            pl.BlockSpec((3, 3, 320, 256), lambda v: (0, 0, 0, 0)),
            pl.BlockSpec((1, 256), lambda v: (0, 0)),
            pl.BlockSpec((3, 3, 256, 256), lambda v: (0, 0, 0, 0)),
            pl.BlockSpec((1, 256), lambda v: (0, 0)),
            pl.BlockSpec((256, 128), lambda v: (0, 0)),
            pl.BlockSpec((1, 128), lambda v: (0, 0)),
        ],
        out_specs=[
            pl.BlockSpec((1, D, NPIX), lambda v: (v, 0, 0)),
            pl.BlockSpec((1, NPIX, IMG_C), lambda v: (v, 0, 0)),
        ],
        out_shape=[
            jax.ShapeDtypeStruct((6, D, NPIX), jnp.float32),
            jax.ShapeDtypeStruct((6, NPIX, IMG_C), jnp.float32),
        ],
        scratch_shapes=[pltpu.VMEM((FH + 2, FW + 2, 256), jnp.float32)],
    )(xpad, w1, b1, w2, b2, w3, b3)


_FT = 16


def _final_body(xp_ref, wf_ref, out_ref):
    f32 = jnp.float32
    bf16 = jnp.bfloat16
    t = pl.program_id(0)
    xt = xp_ref[0, pl.ds(t * _FT, _FT + 2), :, :] + \
        xp_ref[1, pl.ds(t * _FT, _FT + 2), :, :]
    acc = jnp.zeros((_FT * NXY, IMG_C), f32)
    for dy in range(3):
        for dx in range(3):
            a = xt[dy:dy + _FT, dx:dx + NXY, :].reshape(_FT * NXY, IMG_C)
            acc += jnp.dot(a.astype(bf16), wf_ref[dy, dx].astype(bf16),
                           preferred_element_type=f32)
    out_ref[...] = jax.nn.relu(acc)


@jax.jit
def _final_conv(parts, wf):
    xp = jnp.pad(parts.reshape(2, NXX, NXY, IMG_C),
                 ((0, 0), (1, 1), (1, 1), (0, 0)))
    return pl.pallas_call(
        _final_body,
        grid=(NXX // _FT,),
        in_specs=[
            pl.BlockSpec((2, NXX + 2, NXY + 2, IMG_C), lambda t: (0, 0, 0, 0)),
            pl.BlockSpec((3, 3, IMG_C, IMG_C), lambda t: (0, 0, 0, 0)),
        ],
        out_specs=pl.BlockSpec((_FT * NXY, IMG_C), lambda t: (t, 0)),
        out_shape=jax.ShapeDtypeStruct((NVOX, IMG_C), jnp.float32),
    )(xp, wf)


def kernel(image_fpn, depth, camera_intrinsics, camera2lidar, img_aug_matrix,
           lidar_aug_matrix, dt_w1, dt_b1, dt_w2, dt_b2, dt_w3, dt_b3,
           dn_w1, dn_b1, dn_w2, dn_b2, dn_w3, dn_b3, ds_w1):
    relu = jax.nn.relu
    d = depth.reshape(6, 1, IH, IW)
    d = relu(_conv2d(d, dt_w1, dt_b1, 1, 0))
    d = relu(_conv2d(d, dt_w2, dt_b2, 4, 2))
    d = relu(_conv2d(d, dt_w3, dt_b3, 2, 2))
    x = jnp.concatenate([d.transpose(0, 2, 3, 1),
                         image_fpn.transpose(0, 2, 3, 1)], axis=-1)
    xpad = jnp.pad(x, ((0, 0), (1, 1), (1, 1), (0, 0)))
    w1 = dn_w1.transpose(2, 3, 1, 0)
    w2 = dn_w2.transpose(2, 3, 1, 0)
    w3 = jnp.pad(dn_w3[:, :, 0, 0].T, ((0, 0), (0, 5)))
    b3 = jnp.pad(dn_b3, (0, 5))
    dep6, feat2d = _dn_chain(xpad, w1, dn_b1[None], w2, dn_b2[None],
                             w3, b3[None])
    dep = dep6

    intrins = camera_intrinsics[..., :3, :3]
    post_rots = img_aug_matrix[..., :3, :3]
    post_trans = img_aug_matrix[..., :3, 3]
    c2l_r = camera2lidar[..., :3, :3]
    c2l_t = camera2lidar[..., :3, 3]
    fr = _create_frustum()
    pts = fr[None, None] - post_trans[:, :, None, None, None, :]
    pts = jnp.einsum('bnij,bndhwj->bndhwi', jnp.linalg.inv(post_rots), pts)
    pts = jnp.concatenate([pts[..., :2] * pts[..., 2:3], pts[..., 2:3]], -1)
    combine = c2l_r @ jnp.linalg.inv(intrins)
    pts = jnp.einsum('bnij,bndhwj->bndhwi', combine, pts) + c2l_t[:, :, None, None, None, :]
    pts = (jnp.einsum('bij,bndhwj->bndhwi', lidar_aug_matrix[..., :3, :3], pts)
           + lidar_aug_matrix[..., :3, 3][:, None, None, None, None, :])
    g = ((pts - jnp.asarray(BX_NP - DX_NP / 2.0)) / jnp.asarray(DX_NP)).astype(jnp.int32).reshape(6, D, NPIX, 3)
    kept = ((g[..., 0] >= 0) & (g[..., 0] < NXX) & (g[..., 1] >= 0) &
            (g[..., 1] < NXY) & (g[..., 2] >= 0) & (g[..., 2] < NXZ))
    gx = jnp.where(kept, g[..., 0], 0)
    gy = jnp.where(kept, g[..., 1], 0)
    vox = gx * NXY + gy

    dep_w = dep * kept.astype(jnp.float32)
    zeros = jnp.zeros((NVOX // 2, 128), jnp.float32)

    parts = _sc_bev_pool(dep_w, vox, feat2d, zeros)

    wf = ds_w1.transpose(2, 3, 1, 0)
    res = _final_conv(parts, wf)
    out = res.reshape(NXX, NXY, IMG_C).transpose(2, 0, 1)[None]
    return out
